# baseline (speedup 1.0000x reference)
import functools

import jax
import jax.numpy as jnp
from jax import lax
from jax.experimental import pallas as pl
from jax.experimental.pallas import tpu as pltpu
from jax.experimental.pallas import tpu_sc as plsc

VOCAB = 1000000
DIM = 64
BSZ = 16384
SEQLEN = 200

NC = 2
NS = 16
NW = NC * NS

BBLK = 128
NBLK = SEQLEN * (BSZ // BBLK)
PER_W = NBLK // NW
NBT = BSZ // BBLK


def _make_kernel():
    mesh = plsc.VectorSubcoreMesh(core_axis_name="c", subcore_axis_name="s")

    @functools.partial(
        pl.kernel,
        mesh=mesh,
        out_type=jax.ShapeDtypeStruct((SEQLEN, 8, NBT, 8, 128), jnp.float32),
        scratch_types=[
            pltpu.VMEM((BBLK,), jnp.int32),
            pltpu.VMEM((BBLK,), jnp.int32),
            pltpu.VMEM((BBLK, DIM), jnp.float32),
            pltpu.VMEM((BBLK, DIM), jnp.float32),
            pltpu.VMEM((8, 8, 128), jnp.float32),
            pltpu.VMEM((8, 8, 128), jnp.float32),
            pltpu.SemaphoreType.DMA,
            pltpu.SemaphoreType.DMA,
            pltpu.SemaphoreType.DMA,
            pltpu.SemaphoreType.DMA,
        ],
        compiler_params=pltpu.CompilerParams(
            use_tc_tiling_on_sc=False, needs_layout_passes=False),
    )
    def emb_kernel(xt_hbm, table_hbm, out_hbm,
                   idx0, idx1, rows0, rows1, t0, t1, g0, g1, o0, o1):
        wid = lax.axis_index("s") * NC + lax.axis_index("c")
        idx_b = (idx0, idx1)
        rows_b = (rows0, rows1)
        tile_b = (t0, t1)
        gsem = (g0, g1)
        osem = (o0, o1)
        iota = lax.iota(jnp.int32, 16)

        def sbt(i):
            g = i * NW + wid
            s = g // NBT
            bt = g - s * NBT
            return s, bt

        def gstart(i, b):
            s, bt = sbt(i)
            pltpu.sync_copy(xt_hbm.at[s, pl.ds(bt * BBLK, BBLK)], idx_b[b])
            pltpu.async_copy(table_hbm.at[idx_b[b]], rows_b[b], gsem[b])

        def gwait(b):
            pltpu.make_async_copy(
                table_hbm.at[idx_b[b]], rows_b[b], gsem[b]).wait()

        def transpose(b):
            rows = rows_b[b]
            tiles = tile_b[b]

            def dt_body(dt, carry):
                for ds in range(8):
                    d = dt * 8 + ds
                    col = jnp.zeros((16,), jnp.int32) + d
                    for c in range(8):
                        v = plsc.load_gather(rows, [iota + c * 16, col])
                        tiles[dt, ds, pl.ds(c * 16, 16)] = v
                return carry

            lax.fori_loop(0, 8, dt_body, 0)

        def ostart(i, b):
            s, bt = sbt(i)
            pltpu.async_copy(tile_b[b], out_hbm.at[s, :, bt], osem[b])

        def owait(i, b):
            s, bt = sbt(i)
            pltpu.make_async_copy(
                tile_b[b], out_hbm.at[s, :, bt], osem[b]).wait()

        gstart(0, 0)

        def body(j, carry):
            for b in range(2):
                i = 2 * j + b
                nb = 1 - b
                pl.when(i >= 2)(lambda: owait(i - 2, b))
                pl.when(i + 1 < PER_W)(lambda: gstart(i + 1, nb))
                gwait(b)
                transpose(b)
                ostart(i, b)
            return carry

        lax.fori_loop(0, PER_W // 2, body, 0)
        owait(PER_W - 2, 0)
        owait(PER_W - 1, 1)

    return emb_kernel


_EMB = _make_kernel()


def kernel(x, weight):
    out5 = _EMB(x.T, weight)
    return out5.transpose(2, 4, 0, 1, 3).reshape(BSZ, SEQLEN, DIM)
